# Initial kernel scaffold; baseline (speedup 1.0000x reference)
import functools

import jax
import jax.numpy as jnp
from jax import lax
from jax.experimental import pallas as pl
from jax.experimental.pallas import tpu as pltpu
from jax.experimental.pallas import tpu_sc as plsc

VOCAB = 100000
D_MODEL = 64
MAX_SEQ = 200
BATCH = 4096

_NC = 2
_NS = 16
_L = 16
_NW = _NC * _NS

_B_PER_W = BATCH // _NW
_CHUNK_B = 2
_ROWS = _CHUNK_B * MAX_SEQ
_N_CHUNK = _B_PER_W // _CHUNK_B
_IDX_MINOR = 100
_GATHERS = _ROWS // _IDX_MINOR


def _pe_table():
    even_i = jnp.arange(0, D_MODEL, 2).astype(jnp.float32)
    denominator = jnp.power(10000.0, even_i / D_MODEL)
    position = jnp.arange(MAX_SEQ).reshape(MAX_SEQ, 1).astype(jnp.float32)
    even_pe = jnp.sin(position / denominator)
    odd_pe = jnp.cos(position / denominator)
    pe = jnp.stack([even_pe, odd_pe], axis=2).reshape(MAX_SEQ, D_MODEL)
    return jnp.tile(pe, (_CHUNK_B, 1))


def kernel(x, embedding):
    idx = x.reshape(-1, _IDX_MINOR)
    pe = _pe_table()

    mesh = plsc.VectorSubcoreMesh(core_axis_name="c", subcore_axis_name="s")

    @functools.partial(
        pl.kernel,
        mesh=mesh,
        out_type=jax.ShapeDtypeStruct((BATCH * MAX_SEQ, D_MODEL), jnp.float32),
        scratch_types=[
            pltpu.VMEM((_GATHERS, _IDX_MINOR), jnp.int32),
            pltpu.VMEM((_ROWS, D_MODEL), jnp.float32),
            pltpu.VMEM((_ROWS, D_MODEL), jnp.float32),
            pltpu.SemaphoreType.DMA,
        ],
    )
    def sc_kernel(idx_hbm, emb_hbm, pe_hbm, out_hbm, idx_v, buf_v, pe_v, sem):
        wid = lax.axis_index("s") * _NC + lax.axis_index("c")
        pltpu.sync_copy(pe_hbm, pe_v)

        def chunk_body(c, carry):
            base = wid * _N_CHUNK + c
            row0 = base * _ROWS
            blk = base * _GATHERS
            pltpu.sync_copy(idx_hbm.at[pl.ds(blk, _GATHERS)], idx_v)
            copies = [
                pltpu.async_copy(
                    emb_hbm.at[idx_v.at[j]],
                    buf_v.at[pl.ds(j * _IDX_MINOR, _IDX_MINOR)],
                    sem,
                )
                for j in range(_GATHERS)
            ]
            for cpy in copies:
                cpy.wait()

            def add_body(r, c2):
                for j in range(D_MODEL // _L):
                    s = pl.ds(j * _L, _L)
                    buf_v[r, s] = buf_v[r, s] + pe_v[r, s]
                return c2

            lax.fori_loop(0, _ROWS, add_body, 0)
            pltpu.sync_copy(buf_v, out_hbm.at[pl.ds(row0, _ROWS)])
            return carry

        lax.fori_loop(0, _N_CHUNK, chunk_body, 0)

    out = sc_kernel(idx, embedding, pe)
    return out.reshape(BATCH, MAX_SEQ, D_MODEL)


# speedup vs baseline: 3.3515x; 3.3515x over previous
import functools

import jax
import jax.numpy as jnp
from jax import lax
from jax.experimental import pallas as pl
from jax.experimental.pallas import tpu as pltpu
from jax.experimental.pallas import tpu_sc as plsc

VOCAB = 100000
D_MODEL = 64
MAX_SEQ = 200
BATCH = 4096

_NC = 2
_NS = 16
_L = 16
_NW = _NC * _NS

_B_PER_W = BATCH // _NW
_CHUNK_B = 2
_ROWS = _CHUNK_B * MAX_SEQ
_N_CHUNK = _B_PER_W // _CHUNK_B
_IDX_MINOR = 100
_GATHERS = _ROWS // _IDX_MINOR


def _pe_table():
    even_i = jnp.arange(0, D_MODEL, 2).astype(jnp.float32)
    denominator = jnp.power(10000.0, even_i / D_MODEL)
    position = jnp.arange(MAX_SEQ).reshape(MAX_SEQ, 1).astype(jnp.float32)
    even_pe = jnp.sin(position / denominator)
    odd_pe = jnp.cos(position / denominator)
    pe = jnp.stack([even_pe, odd_pe], axis=2).reshape(MAX_SEQ, D_MODEL)
    return jnp.tile(pe, (_CHUNK_B, 1))


def kernel(x, embedding):
    idx = x.reshape(-1, _IDX_MINOR)
    pe = _pe_table()

    mesh = plsc.VectorSubcoreMesh(core_axis_name="c", subcore_axis_name="s")

    @functools.partial(
        pl.kernel,
        mesh=mesh,
        out_type=jax.ShapeDtypeStruct((BATCH * MAX_SEQ, D_MODEL), jnp.float32),
        compiler_params=pltpu.CompilerParams(use_tc_tiling_on_sc=False),
        scratch_types=[
            pltpu.VMEM((_GATHERS, _IDX_MINOR), jnp.int32),
            pltpu.VMEM((_ROWS, D_MODEL), jnp.float32),
            pltpu.VMEM((_ROWS, D_MODEL), jnp.float32),
            pltpu.SemaphoreType.DMA,
        ],
    )
    def sc_kernel(idx_hbm, emb_hbm, pe_hbm, out_hbm, idx_v, buf_v, pe_v, sem):
        wid = lax.axis_index("s") * _NC + lax.axis_index("c")
        pltpu.sync_copy(pe_hbm, pe_v)

        def chunk_body(c, carry):
            base = wid * _N_CHUNK + c
            row0 = base * _ROWS
            blk = base * _GATHERS
            pltpu.sync_copy(idx_hbm.at[pl.ds(blk, _GATHERS)], idx_v)
            copies = [
                pltpu.async_copy(
                    emb_hbm.at[idx_v.at[j]],
                    buf_v.at[pl.ds(j * _IDX_MINOR, _IDX_MINOR)],
                    sem,
                )
                for j in range(_GATHERS)
            ]
            for cpy in copies:
                cpy.wait()

            def add_body(r, c2):
                for j in range(D_MODEL // _L):
                    s = pl.ds(j * _L, _L)
                    buf_v[r, s] = buf_v[r, s] + pe_v[r, s]
                return c2

            lax.fori_loop(0, _ROWS, add_body, 0)
            pltpu.sync_copy(buf_v, out_hbm.at[pl.ds(row0, _ROWS)])
            return carry

        lax.fori_loop(0, _N_CHUNK, chunk_body, 0)

    out = sc_kernel(idx, embedding, pe)
    return out.reshape(BATCH, MAX_SEQ, D_MODEL)
